# baseline (speedup 1.0000x reference)
import jax
import jax.numpy as jnp
from jax.experimental import pallas as pl
from jax.experimental.pallas import tpu as pltpu

TOK_BLK = 1024
NBUF = 4


def _router_kernel(x_hbm, w_ref, o_ref, xbuf, sems):
    i = pl.program_id(0)
    nstep = pl.num_programs(0)

    def start(blk):
        slot = jax.lax.rem(blk, NBUF)
        pltpu.make_async_copy(
            x_hbm.at[pl.ds(blk * TOK_BLK, TOK_BLK), :],
            xbuf.at[slot],
            sems.at[slot],
        ).start()

    @pl.when(i == 0)
    def _():
        for b in range(NBUF - 1):
            start(b)

    @pl.when(i + NBUF - 1 < nstep)
    def _():
        start(i + NBUF - 1)

    slot = jax.lax.rem(i, NBUF)
    pltpu.make_async_copy(
        x_hbm.at[pl.ds(i * TOK_BLK, TOK_BLK), :],
        xbuf.at[slot],
        sems.at[slot],
    ).wait()
    o_ref[...] = jax.lax.dot_general(
        xbuf[slot], w_ref[...],
        dimension_numbers=(((1,), (1,)), ((), ())),
        preferred_element_type=jnp.float32)


def kernel(hidden_states, W):
    B, S, H = hidden_states.shape
    E = W.shape[0]
    T = B * S
    x = hidden_states.reshape(T, H)
    out = pl.pallas_call(
        _router_kernel,
        grid=(T // TOK_BLK,),
        in_specs=[
            pl.BlockSpec(memory_space=pltpu.MemorySpace.HBM),
            pl.BlockSpec((E, H), lambda i: (0, 0)),
        ],
        out_specs=pl.BlockSpec((TOK_BLK, E), lambda i: (i, 0)),
        out_shape=jax.ShapeDtypeStruct((T, E), jnp.float32),
        scratch_shapes=[
            pltpu.VMEM((NBUF, TOK_BLK, H), jnp.float32),
            pltpu.SemaphoreType.DMA((NBUF,)),
        ],
        compiler_params=pltpu.CompilerParams(
            dimension_semantics=("arbitrary",),
        ),
    )(x, W)
    return out.reshape(B, S, E)


# speedup vs baseline: 1.0176x; 1.0176x over previous
import jax
import jax.numpy as jnp
from jax.experimental import pallas as pl
from jax.experimental.pallas import tpu as pltpu

TOK_BLK = 1024


def _router_kernel(x_ref, w_ref, o_ref):
    o_ref[...] = jax.lax.dot_general(
        x_ref[...], w_ref[...],
        dimension_numbers=(((1,), (1,)), ((), ())),
        preferred_element_type=jnp.float32)


def kernel(hidden_states, W):
    B, S, H = hidden_states.shape
    E = W.shape[0]
    T = B * S
    x = hidden_states.reshape(T, H)
    out = pl.pallas_call(
        _router_kernel,
        grid=(T // TOK_BLK,),
        in_specs=[
            pl.BlockSpec((TOK_BLK, H), lambda i: (i, 0)),
            pl.BlockSpec((E, H), lambda i: (0, 0)),
        ],
        out_specs=pl.BlockSpec((TOK_BLK, E), lambda i: (i, 0)),
        out_shape=jax.ShapeDtypeStruct((T, E), jnp.float32),
        compiler_params=pltpu.CompilerParams(
            dimension_semantics=("parallel",),
        ),
    )(x, W)
    return out.reshape(B, S, E)
